# Initial kernel scaffold; baseline (speedup 1.0000x reference)
import jax
import jax.numpy as jnp
from jax.experimental import pallas as pl


def kernel(x, rc1_w, rc1_b, rc2_w, rc2_b, exp_w1, exp_b1, exp_w2, exp_b2, sh1_w, sh1_b, sh2_w, sh2_b, ln_g, ln_b):
    raise NotImplementedError("write your pallas kernel here")


# speedup vs baseline: 1.1201x; 1.1201x over previous
import functools

import jax
import jax.numpy as jnp
from jax.experimental import pallas as pl
from jax.experimental.pallas import tpu as pltpu

S, D, H, E, TOPK = 2048, 1024, 1024 * 2, 8, 2
PREC = jax.lax.Precision.DEFAULT
BT = 256
NT = S // BT


def _router_body(xpad_ref, w_ref, b1_ref, v_ref, b2_ref, gates_ref):
    t = pl.program_id(0)
    base = t * BT
    win = xpad_ref[pl.ds(base, BT + 16), :]
    g = None
    for k in range(3):
        xs = win[6 + k:6 + k + BT + 8, :]
        tt = jax.lax.dot_general(xs, w_ref[k], (((1,), (0,)), ((), ())),
                                 precision=PREC,
                                 preferred_element_type=jnp.float32)
        g = tt if g is None else g + tt
    g = g + b1_ref[...]
    g = 0.5 * g * (1.0 + jax.lax.erf(g * 0.7071067811865476))
    rid = jax.lax.broadcasted_iota(jnp.int32, (BT + 8, 1), 0) + base - 1
    g = jnp.where((rid >= 0) & (rid < S), g, 0.0)
    lo = None
    for k in range(3):
        tt = jax.lax.dot_general(g[k:k + BT, :], v_ref[k],
                                 (((1,), (0,)), ((), ())), precision=PREC,
                                 preferred_element_type=jnp.float32)
        lo = tt if lo is None else lo + tt
    lo = lo + b2_ref[...]
    eid = jax.lax.broadcasted_iota(jnp.int32, (BT, E), 1)
    m1 = jnp.max(lo, axis=1, keepdims=True)
    a1 = jnp.min(jnp.where(lo >= m1, eid, E), axis=1, keepdims=True)
    lo2 = jnp.where(eid == a1, -jnp.inf, lo)
    m2 = jnp.max(lo2, axis=1, keepdims=True)
    a2 = jnp.min(jnp.where(lo2 >= m2, eid, E), axis=1, keepdims=True)
    g1 = 1.0 / (1.0 + jnp.exp(m2 - m1))
    g2 = 1.0 - g1
    gates_ref[...] = jnp.where(eid == a1, g1, 0.0) + jnp.where(eid == a2, g2, 0.0)


def _moe_dense_body(x_ref, w1_ref, b1_ref, w2_ref, b2_ref, g_ref, y_ref):
    e = pl.program_id(0)
    t = pl.program_id(1)
    x = x_ref[...]
    h = jax.lax.dot_general(x, w1_ref[0], (((1,), (0,)), ((), ())),
                            precision=PREC, preferred_element_type=jnp.float32)
    h = h + b1_ref[0]
    h = jnp.where(h > 0, h, jnp.exp(jnp.minimum(h, 0.0)) - 1.0)
    o = jax.lax.dot_general(h, w2_ref[0], (((1,), (0,)), ((), ())),
                            precision=PREC, preferred_element_type=jnp.float32)
    o = (o + b2_ref[0]) * g_ref[0]

    @pl.when(e == 0)
    def _():
        y_ref[pl.ds(t * BT, BT), :] = o

    @pl.when(e != 0)
    def _():
        y_ref[pl.ds(t * BT, BT), :] = y_ref[pl.ds(t * BT, BT), :] + o


def _shared1_body(xpad_ref, w_ref, b_ref, h_ref):
    t = pl.program_id(0)
    base = t * BT
    win = xpad_ref[pl.ds(base, BT + 16), :]
    g = None
    for k in range(3):
        xs = win[7 + k:7 + k + BT, :]
        tt = jax.lax.dot_general(xs, w_ref[k], (((1,), (0,)), ((), ())),
                                 precision=PREC,
                                 preferred_element_type=jnp.float32)
        g = tt if g is None else g + tt
    g = g + b_ref[...]
    h_ref[...] = g * jax.nn.sigmoid(g)


def _shared2_body(h_ref, w_ref, b_ref, x_ref, y_ref, lng_ref, lnb_ref, o_ref):
    t = pl.program_id(0)
    base = t * BT
    win = h_ref[pl.ds(base, BT + 8), :]
    sh = None
    for k in range(3):
        hs = win[k:k + BT, :]
        tt = jax.lax.dot_general(hs, w_ref[k], (((1,), (0,)), ((), ())),
                                 precision=PREC,
                                 preferred_element_type=jnp.float32)
        sh = tt if sh is None else sh + tt
    z = x_ref[...] + y_ref[...] + sh + b_ref[...]
    mu = jnp.mean(z, axis=1, keepdims=True)
    zc = z - mu
    var = jnp.mean(zc * zc, axis=1, keepdims=True)
    o_ref[...] = zc * jax.lax.rsqrt(var + 1e-5) * lng_ref[...] + lnb_ref[...]


def kernel(x, rc1_w, rc1_b, rc2_w, rc2_b, exp_w1, exp_b1, exp_w2, exp_b2,
           sh1_w, sh1_b, sh2_w, sh2_b, ln_g, ln_b):
    xf = x.reshape(S, D)
    xpad = jnp.pad(xf, ((8, 8), (0, 0)))
    rc1 = jnp.transpose(rc1_w, (2, 1, 0))
    rc2 = jnp.transpose(rc2_w, (2, 1, 0))
    s1 = jnp.transpose(sh1_w, (2, 1, 0))
    s2 = jnp.transpose(sh2_w, (2, 1, 0))

    whole = lambda *shape: pl.BlockSpec(shape, lambda *a: tuple(0 for _ in shape))

    gates = pl.pallas_call(
        _router_body,
        grid=(NT,),
        in_specs=[
            whole(S + 16, D),
            whole(3, D, D),
            whole(1, D),
            whole(3, D, E),
            whole(1, E),
        ],
        out_specs=pl.BlockSpec((BT, E), lambda t: (t, 0)),
        out_shape=jax.ShapeDtypeStruct((S, E), jnp.float32),
    )(xpad, rc1, rc1_b[None, :], rc2, rc2_b[None, :])

    gcol = jnp.transpose(gates).reshape(E, S, 1)

    y = pl.pallas_call(
        _moe_dense_body,
        grid=(E, NT),
        in_specs=[
            pl.BlockSpec((BT, D), lambda e, t: (t, 0)),
            pl.BlockSpec((1, D, H), lambda e, t: (e, 0, 0)),
            pl.BlockSpec((1, 1, H), lambda e, t: (e, 0, 0)),
            pl.BlockSpec((1, H, D), lambda e, t: (e, 0, 0)),
            pl.BlockSpec((1, 1, D), lambda e, t: (e, 0, 0)),
            pl.BlockSpec((1, BT, 1), lambda e, t: (e, t, 0)),
        ],
        out_specs=pl.BlockSpec((S, D), lambda e, t: (0, 0)),
        out_shape=jax.ShapeDtypeStruct((S, D), jnp.float32),
        compiler_params=pltpu.CompilerParams(
            dimension_semantics=("arbitrary", "arbitrary")),
    )(xf, exp_w1, exp_b1[:, None, :], exp_w2, exp_b2[:, None, :], gcol)

    h = pl.pallas_call(
        _shared1_body,
        grid=(NT,),
        in_specs=[whole(S + 16, D), whole(3, D, H), whole(1, H)],
        out_specs=pl.BlockSpec((BT, H), lambda t: (t, 0)),
        out_shape=jax.ShapeDtypeStruct((S, H), jnp.float32),
    )(xpad, s1, sh1_b[None, :])

    hpad = jnp.pad(h, ((1, 7), (0, 0)))

    out = pl.pallas_call(
        _shared2_body,
        grid=(NT,),
        in_specs=[
            whole(S + 8, H),
            whole(3, H, D),
            whole(1, D),
            pl.BlockSpec((BT, D), lambda t: (t, 0)),
            pl.BlockSpec((BT, D), lambda t: (t, 0)),
            whole(1, D),
            whole(1, D),
        ],
        out_specs=pl.BlockSpec((BT, D), lambda t: (t, 0)),
        out_shape=jax.ShapeDtypeStruct((S, D), jnp.float32),
    )(hpad, s2, sh2_b[None, :], xf, y, ln_g[None, :], ln_b[None, :])

    return out.reshape(1, S, D)
